# baseline (speedup 1.0000x reference)
import functools

import jax
import jax.numpy as jnp
import numpy as np
from jax import lax
from jax.experimental import pallas as pl
from jax.experimental.pallas import tpu as pltpu
from jax.experimental.pallas import tpu_sc as plsc

NUM_CORES = 2
NUM_SUBCORES = 16
NW = NUM_CORES * NUM_SUBCORES

PARAMS_PAD = 96
CBASE = PARAMS_PAD

R_THREE = 0
R_CH1 = 1
R_CH2 = 2
R_B0 = 3
R_T0 = 5
R_W0 = 15
N_CROWS = 25
TOTAL_PAR = CBASE + N_CROWS * 16


def _const_rows():
    ci = np.zeros((N_CROWS, 16), np.int32)
    lane = np.arange(16)
    ci[R_THREE] = 3
    ci[R_CH1] = 1
    ci[R_CH2] = 2
    for j in range(2):
        e = lane + 16 * j
        k, c = e // 3, e % 3
        ci[R_B0 + j] = c + 65
        for d in range(5):
            ci[R_T0 + j * 5 + d] = k * 5 + d
            ci[R_W0 + j * 5 + d] = c * 5 + 50 + d
    return ci.reshape(-1)


@functools.partial(jax.jit, static_argnums=(2, 3, 4))
def _run(xt, params, L, B, rows_per_chunk):
    stripe = B // NW
    cols = stripe // 2
    n_rc = L // rows_per_chunk

    mesh = plsc.VectorSubcoreMesh(core_axis_name="c", subcore_axis_name="s")

    @functools.partial(
        pl.kernel,
        mesh=mesh,
        compiler_params=pltpu.CompilerParams(
            needs_layout_passes=False, use_tc_tiling_on_sc=True
        ),
        out_type=jax.ShapeDtypeStruct((3, L, B), jnp.float32),
        scratch_types=[
            pltpu.VMEM((TOTAL_PAR,), jnp.float32),
            pltpu.VMEM((32,), jnp.float32),
            pltpu.VMEM((2, rows_per_chunk, cols), jnp.int32),
            pltpu.VMEM((2, 3, rows_per_chunk, cols), jnp.float32),
            pltpu.SemaphoreType.DMA,
            pltpu.SemaphoreType.DMA,
            pltpu.SemaphoreType.DMA,
            pltpu.SemaphoreType.DMA,
        ],
    )
    def sc_kernel(xt_hbm, params_hbm, out_hbm,
                  params_v, lut_v, xv2, ov2, is0, is1, os0, os1):
        wid = lax.axis_index("s") * NUM_CORES + lax.axis_index("c")
        col0 = wid * stripe
        in_sems = (is0, is1)
        out_sems = (os0, os1)

        def crow(r):
            return plsc.bitcast(
                params_v[pl.ds(CBASE + 16 * r, 16)], jnp.int32)

        def in_copy(r, h):
            return pltpu.make_async_copy(
                xt_hbm.at[pl.ds(r * rows_per_chunk, rows_per_chunk),
                          pl.ds(col0 + h * cols, cols)],
                xv2.at[h], in_sems[h])

        def out_copy(c, r, h):
            return pltpu.make_async_copy(
                ov2.at[h, c],
                out_hbm.at[c, pl.ds(r * rows_per_chunk, rows_per_chunk),
                           pl.ds(col0 + h * cols, cols)],
                out_sems[h])

        def compute(b):
            @plsc.parallel_loop(0, rows_per_chunk)
            def row_body(r):
                for cb in range(cols // 16):
                    x16 = xv2[b, r, pl.ds(cb * 16, 16)]
                    x3 = x16 * three
                    ov2[b, 0, r, pl.ds(cb * 16, 16)] = plsc.load_gather(
                        lut_v, [x3])
                    ov2[b, 1, r, pl.ds(cb * 16, 16)] = plsc.load_gather(
                        lut_v, [x3 + ch1])
                    ov2[b, 2, r, pl.ds(cb * 16, 16)] = plsc.load_gather(
                        lut_v, [x3 + ch2])

        def step(r, h, first):
            in_copy(r, h).wait()
            if not first:
                for c in range(3):
                    out_copy(c, r - 1, h).wait()
            compute(h)

            @pl.when(r + 1 < n_rc)
            def _():
                in_copy(r + 1, h).start()

            for c in range(3):
                out_copy(c, r, h).start()

        in_copy(0, 0).start()
        in_copy(0, 1).start()

        pltpu.sync_copy(params_hbm, params_v)
        one = params_v[pl.ds(80, 16)]
        for j in range(2):
            acc = plsc.load_gather(params_v, [crow(R_B0 + j)])
            for d in range(5):
                tv = plsc.load_gather(params_v, [crow(R_T0 + j * 5 + d)])
                wv = plsc.load_gather(params_v, [crow(R_W0 + j * 5 + d)])
                acc = acc + tv * wv
            sig = one / (one + jnp.exp(-acc))
            lut_v[pl.ds(16 * j, 16)] = sig

        three = crow(R_THREE)
        ch1 = crow(R_CH1)
        ch2 = crow(R_CH2)

        step(0, 0, True)
        step(0, 1, True)

        def rc_body(t, _):
            r = 1 + t
            step(r, 0, False)
            step(r, 1, False)
            return 0

        lax.fori_loop(0, n_rc - 1, rc_body, 0)

        for h in (0, 1):
            for c in range(3):
                out_copy(c, n_rc - 1, h).wait()

    return sc_kernel(xt, params)


def kernel(x, table, W, b):
    B, L = x.shape
    xt = jnp.transpose(x.astype(jnp.int32))
    tail = np.zeros((12 + 16 + N_CROWS * 16,), np.float32)
    tail[12:28] = 1.0
    tail[28:] = _const_rows().view(np.float32)
    params = jnp.concatenate([
        table.reshape(-1).astype(jnp.float32),
        W.reshape(-1).astype(jnp.float32),
        b.astype(jnp.float32),
        jnp.asarray(tail),
    ])
    out_t = _run(xt, params, L, B, 40)
    return jnp.transpose(out_t, (2, 1, 0))


# speedup vs baseline: 1.0582x; 1.0582x over previous
import functools

import jax
import jax.numpy as jnp
import numpy as np
from jax import lax
from jax.experimental import pallas as pl
from jax.experimental.pallas import tpu as pltpu
from jax.experimental.pallas import tpu_sc as plsc

NUM_CORES = 2
NUM_SUBCORES = 16
NW = NUM_CORES * NUM_SUBCORES

PARAMS_PAD = 96
CBASE = PARAMS_PAD

R_THREE = 0
R_CH1 = 1
R_CH2 = 2
R_B0 = 3
R_T0 = 5
R_W0 = 15
N_CROWS = 25
TOTAL_PAR = CBASE + N_CROWS * 16


def _const_rows():
    ci = np.zeros((N_CROWS, 16), np.int32)
    lane = np.arange(16)
    ci[R_THREE] = 3
    ci[R_CH1] = 1
    ci[R_CH2] = 2
    for j in range(2):
        e = lane + 16 * j
        k, c = e // 3, e % 3
        ci[R_B0 + j] = c + 65
        for d in range(5):
            ci[R_T0 + j * 5 + d] = k * 5 + d
            ci[R_W0 + j * 5 + d] = c * 5 + 50 + d
    return ci.reshape(-1)


@functools.partial(jax.jit, static_argnums=(2, 3, 4))
def _run(xt, params, L, B, rows_per_chunk):
    stripe = B // NW
    cols = stripe // 2
    n_rc = L // rows_per_chunk

    mesh = plsc.VectorSubcoreMesh(core_axis_name="c", subcore_axis_name="s")

    @functools.partial(
        pl.kernel,
        mesh=mesh,
        compiler_params=pltpu.CompilerParams(
            needs_layout_passes=False, use_tc_tiling_on_sc=True
        ),
        out_type=jax.ShapeDtypeStruct((3, L, B), jnp.float32),
        scratch_types=[
            pltpu.VMEM((TOTAL_PAR,), jnp.float32),
            pltpu.VMEM((32,), jnp.float32),
            pltpu.VMEM((2, rows_per_chunk, cols), jnp.int32),
            pltpu.VMEM((2, 3, rows_per_chunk, cols), jnp.float32),
            pltpu.SemaphoreType.DMA,
            pltpu.SemaphoreType.DMA,
            pltpu.SemaphoreType.DMA,
            pltpu.SemaphoreType.DMA,
        ],
    )
    def sc_kernel(xt_hbm, params_hbm, out_hbm,
                  params_v, lut_v, xv2, ov2, is0, is1, os0, os1):
        wid = lax.axis_index("s") * NUM_CORES + lax.axis_index("c")
        col0 = wid * stripe
        in_sems = (is0, is1)
        out_sems = (os0, os1)

        def crow(r):
            return params_v[pl.ds(CBASE + 16 * r, 16)].astype(jnp.int32)

        def in_copy(r, h):
            return pltpu.make_async_copy(
                xt_hbm.at[pl.ds(r * rows_per_chunk, rows_per_chunk),
                          pl.ds(col0 + h * cols, cols)],
                xv2.at[h], in_sems[h])

        def out_copy(c, r, h):
            return pltpu.make_async_copy(
                ov2.at[h, c],
                out_hbm.at[c, pl.ds(r * rows_per_chunk, rows_per_chunk),
                           pl.ds(col0 + h * cols, cols)],
                out_sems[h])

        def compute(b):
            @plsc.parallel_loop(0, rows_per_chunk)
            def row_body(r):
                for cb in range(cols // 16):
                    x16 = xv2[b, r, pl.ds(cb * 16, 16)]
                    x3 = x16 * three
                    ov2[b, 0, r, pl.ds(cb * 16, 16)] = plsc.load_gather(
                        lut_v, [x3])
                    ov2[b, 1, r, pl.ds(cb * 16, 16)] = plsc.load_gather(
                        lut_v, [x3 + ch1])
                    ov2[b, 2, r, pl.ds(cb * 16, 16)] = plsc.load_gather(
                        lut_v, [x3 + ch2])

        def step(r, h, first):
            in_copy(r, h).wait()
            if not first:
                for c in range(3):
                    out_copy(c, r - 1, h).wait()
            compute(h)

            @pl.when(r + 1 < n_rc)
            def _():
                in_copy(r + 1, h).start()

            for c in range(3):
                out_copy(c, r, h).start()

        in_copy(0, 0).start()
        in_copy(0, 1).start()

        pltpu.sync_copy(params_hbm, params_v)
        one = params_v[pl.ds(80, 16)]
        for j in range(2):
            acc = plsc.load_gather(params_v, [crow(R_B0 + j)])
            for d in range(5):
                tv = plsc.load_gather(params_v, [crow(R_T0 + j * 5 + d)])
                wv = plsc.load_gather(params_v, [crow(R_W0 + j * 5 + d)])
                acc = acc + tv * wv
            sig = one / (one + jnp.exp(-acc))
            lut_v[pl.ds(16 * j, 16)] = sig

        three = crow(R_THREE)
        ch1 = crow(R_CH1)
        ch2 = crow(R_CH2)

        step(0, 0, True)
        step(0, 1, True)

        def rc_body(t, _):
            r = 1 + t
            step(r, 0, False)
            step(r, 1, False)
            return 0

        lax.fori_loop(0, n_rc - 1, rc_body, 0)

        for h in (0, 1):
            for c in range(3):
                out_copy(c, n_rc - 1, h).wait()

    return sc_kernel(xt, params)


def kernel(x, table, W, b):
    B, L = x.shape
    xt = jnp.transpose(x.astype(jnp.int32))
    tail = np.zeros((12 + 16 + N_CROWS * 16,), np.float32)
    tail[12:28] = 1.0
    tail[28:] = _const_rows().astype(np.float32)
    params = jnp.concatenate([
        table.reshape(-1).astype(jnp.float32),
        W.reshape(-1).astype(jnp.float32),
        b.astype(jnp.float32),
        jnp.asarray(tail),
    ])
    out_t = _run(xt, params, L, B, 40)
    return jnp.transpose(out_t, (2, 1, 0))
